# Initial kernel scaffold; baseline (speedup 1.0000x reference)
import jax
import jax.numpy as jnp
from jax.experimental import pallas as pl


def kernel(feat_map, temperature):
    raise NotImplementedError("write your pallas kernel here")


# speedup vs baseline: 1.3900x; 1.3900x over previous
import jax
import jax.numpy as jnp
import numpy as np
from jax.experimental import pallas as pl
from jax.experimental.pallas import tpu as pltpu

_B, _D, _H, _W = 64, 256, 5, 5
_HW = _H * _W

_IU, _JU = np.triu_indices(_D)
_TRIU_FLAT = (_IU * _D + _JU).astype(np.int32)


def _bdc_kernel(fc_ref, ft_ref, temp_ref, out_ref):
    fc = fc_ref[0]
    ft = ft_ref[0]
    ac = jnp.abs(fc)
    ar = jnp.abs(ft)
    sc = jnp.sign(fc)
    sr = jnp.sign(ft)
    acc = jnp.zeros((_D, _D), jnp.float32)
    for p in range(_HW):
        m = jnp.minimum(ac[:, p : p + 1], ar[p : p + 1, :])
        s = sc[:, p : p + 1] * sr[p : p + 1, :]
        acc = acc + m * s
    row_mean = jnp.sum(acc, axis=1, keepdims=True) * (1.0 / _D)
    col_mean = jnp.sum(acc, axis=0, keepdims=True) * (1.0 / _D)
    scale = jnp.exp(temp_ref[...])
    out_ref[0] = (acc - row_mean - col_mean) * scale


def kernel(feat_map, temperature):
    b, d, h, w = feat_map.shape
    fc = feat_map.reshape(b, d, h * w)
    ft = fc.transpose(0, 2, 1)
    full = pl.pallas_call(
        _bdc_kernel,
        grid=(b,),
        in_specs=[
            pl.BlockSpec((1, d, h * w), lambda i: (i, 0, 0)),
            pl.BlockSpec((1, h * w, d), lambda i: (i, 0, 0)),
            pl.BlockSpec((1, 1), lambda i: (0, 0)),
        ],
        out_specs=pl.BlockSpec((1, d, d), lambda i: (i, 0, 0)),
        out_shape=jax.ShapeDtypeStruct((b, d, d), jnp.float32),
        compiler_params=pltpu.CompilerParams(
            dimension_semantics=("parallel",),
        ),
        name="bdc_pool",
    )(fc, ft, temperature)
    return jnp.take(full.reshape(b, d * d), _TRIU_FLAT, axis=1)


# speedup vs baseline: 2.7530x; 1.9806x over previous
import jax
import jax.numpy as jnp
import numpy as np
from jax.experimental import pallas as pl
from jax.experimental.pallas import tpu as pltpu

_B, _D, _H, _W = 64, 256, 5, 5
_HW = _H * _W
_TRI = _D * (_D + 1) // 2


def _bdc_kernel(fc_ref, ft_ref, temp_ref, out_ref, scr):
    fc = fc_ref[0]
    ft = ft_ref[0]
    ra = jnp.abs(ft)
    rn = -ra
    rs2 = jnp.where(ft < 0.0, -2.0, 2.0)
    acc = jnp.zeros((_D, _D), jnp.float32)
    for p in range(_HW):
        c = fc[:, p : p + 1]
        t = jnp.minimum(jnp.maximum(c, rn[p : p + 1, :]), ra[p : p + 1, :])
        acc = acc + rs2[p : p + 1, :] * t
    row_mean = jnp.sum(acc, axis=1, keepdims=True) * (1.0 / _D)
    col_mean = jnp.sum(acc, axis=0, keepdims=True) * (1.0 / _D)
    scale = 0.5 * jnp.exp(temp_ref[...])
    scr[...] = (acc - row_mean - col_mean) * scale
    for i in range(_D):
        off = i * _D - (i * (i - 1)) // 2
        out_ref[0, 0, pl.ds(off, _D - i)] = scr[i, i:]


def kernel(feat_map, temperature):
    b, d, h, w = feat_map.shape
    fc = feat_map.reshape(b, d, h * w)
    ft = fc.transpose(0, 2, 1)
    packed = pl.pallas_call(
        _bdc_kernel,
        grid=(b,),
        in_specs=[
            pl.BlockSpec((1, d, h * w), lambda i: (i, 0, 0)),
            pl.BlockSpec((1, h * w, d), lambda i: (i, 0, 0)),
            pl.BlockSpec((1, 1), lambda i: (0, 0)),
        ],
        out_specs=pl.BlockSpec((1, 1, _TRI), lambda i: (i, 0, 0)),
        out_shape=jax.ShapeDtypeStruct((b, 1, _TRI), jnp.float32),
        scratch_shapes=[pltpu.VMEM((_D, _D), jnp.float32)],
        compiler_params=pltpu.CompilerParams(
            dimension_semantics=("parallel",),
        ),
        name="bdc_pool",
    )(fc, ft, temperature)
    return packed.reshape(b, _TRI)
